# Initial kernel scaffold; baseline (speedup 1.0000x reference)
import jax
import jax.numpy as jnp
from jax.experimental import pallas as pl


def kernel(embeddings, margins, slabels, clabels, batch_idx):
    raise NotImplementedError("write your pallas kernel here")


# speedup vs baseline: 7.2359x; 7.2359x over previous
import functools

import jax
import jax.numpy as jnp
from jax import lax
from jax.experimental import pallas as pl
from jax.experimental.pallas import tpu as pltpu

N = 131072
D = 8
NB_PTS = 1024
NUM_BLK = N // NB_PTS
NSEG = 512
NROW = 640
BROW = 544

XC_LO = 1.0000005000002917e-06
XC_HI = 13.815510557964274

_DN = (((1,), (1,)), ((), ()))
_DN2 = (((1,), (0,)), ((), ()))
_PREC = lax.Precision.HIGHEST


def _stats_kernel(et_ref, mt_ref, sl_ref, cl_ref, bi_ref, out_ref):
    step = pl.program_id(0)
    sl = sl_ref[0]
    cl = cl_ref[0]
    bi = bi_ref[0]
    sid = bi * 128 + sl * 32 + cl
    sid = jnp.where(sl < 4, sid, 560)
    rows = lax.broadcasted_iota(jnp.int32, (NROW, NB_PTS), 0)
    onehot = ((rows == sid) | (rows == BROW + bi)).astype(jnp.float32)
    e = et_ref[...]
    m = mt_ref[...]
    ones = jnp.ones((1, NB_PTS), jnp.float32)
    vals = jnp.concatenate([e, m, ones], axis=0)
    partial = lax.dot_general(onehot, vals, _DN,
                              preferred_element_type=jnp.float32,
                              precision=_PREC)

    @pl.when(step == 0)
    def _():
        out_ref[...] = jnp.zeros_like(out_ref)

    out_ref[...] += partial


def _loss_kernel(stats_ref, et_ref, mt_ref, sl_ref, cl_ref, bi_ref,
                 out_ref, bce_acc, sm_acc):
    step = pl.program_id(0)

    cnt = stats_ref[0:NSEG, 9:10]
    cnt_safe = jnp.maximum(cnt, 1.0)
    inv_cnt = 1.0 / cnt_safe
    cmat = stats_ref[0:NSEG, 0:D] * inv_cnt
    sigma = stats_ref[0:NSEG, 8:9] * inv_cnt
    inv_den = 1.0 / (2.0 * sigma * sigma + 1e-8)
    cnorm2 = jnp.sum(cmat * cmat, axis=1, keepdims=True)

    e = et_ref[...]
    m = mt_ref[...]
    sl = sl_ref[0]
    cl = cl_ref[0]
    bi = bi_ref[0]
    sid = bi * 128 + sl * 32 + cl
    gid = bi * 4 + sl

    enorm2 = jnp.sum(e * e, axis=0, keepdims=True)
    dotp = lax.dot_general(cmat, e, _DN2,
                           preferred_element_type=jnp.float32,
                           precision=_PREC)
    x = (cnorm2 - 2.0 * dotp + enorm2) * inv_den
    xc = jnp.clip(x, XC_LO, XC_HI)
    p = jnp.exp(-xc)
    log1mp = jnp.log1p(-p)

    rows = lax.broadcasted_iota(jnp.int32, (NSEG, NB_PTS), 0)
    tgt = rows == sid
    g_lo = gid * 32
    valid = (rows >= g_lo) & (rows < g_lo + 32) & (sl < 4)
    term = jnp.where(tgt, xc, -log1mp)
    term = jnp.where(valid, term, 0.0)
    bce_partial = jnp.sum(term, axis=1, keepdims=True)

    dm = m - sigma
    smooth = jnp.where(tgt & valid, dm * dm, 0.0)
    sm_partial = jnp.sum(smooth, axis=1, keepdims=True)

    @pl.when(step == 0)
    def _():
        bce_acc[...] = jnp.zeros_like(bce_acc)
        sm_acc[...] = jnp.zeros_like(sm_acc)

    bce_acc[...] += bce_partial
    sm_acc[...] += sm_partial

    @pl.when(step == NUM_BLK - 1)
    def _():
        gi = lax.broadcasted_iota(jnp.int32, (16, NSEG), 0)
        si = lax.broadcasted_iota(jnp.int32, (16, NSEG), 1)
        m1 = ((si // 32) == gi).astype(jnp.float32)
        bi4 = lax.broadcasted_iota(jnp.int32, (4, 16), 0)
        gi16 = lax.broadcasted_iota(jnp.int32, (4, 16), 1)
        m2 = ((gi16 // 4) == bi4).astype(jnp.float32)

        def gdot(mat, vec, dn=_DN2):
            return lax.dot_general(mat, vec, dn,
                                   preferred_element_type=jnp.float32,
                                   precision=_PREC)

        present = (cnt > 0.0).astype(jnp.float32)
        n_sel = gdot(m1, cnt)
        npres = gdot(m1, present)
        bce_g = gdot(m1, present * bce_acc[...])
        sm_g = gdot(m1, sm_acc[...])
        n_sel_safe = jnp.maximum(n_sel, 1.0)
        npres_safe = jnp.maximum(npres, 1.0)
        ml = bce_g / n_sel_safe / npres_safe
        sml = sm_g / npres_safe
        s_present = (n_sel > 0.0).astype(jnp.float32)
        contrib = s_present * (ml + sml)
        cls_sum = gdot(m2, contrib)
        cls_cnt = gdot(m2, s_present)
        batch_loss = cls_sum / jnp.maximum(cls_cnt, 1.0)
        bcnt = stats_ref[BROW:BROW + 4, 9:10]
        b_present = (bcnt > 0.0).astype(jnp.float32)
        num = jnp.sum(b_present * batch_loss, keepdims=True)
        den = jnp.maximum(jnp.sum(b_present, keepdims=True), 1.0)
        out_ref[...] = num / den


def _int_blocks(x):
    return x.astype(jnp.int32).reshape(NUM_BLK, 1, NB_PTS)


@jax.jit
def kernel(embeddings, margins, slabels, clabels, batch_idx):
    et = embeddings.T
    mt = margins.reshape(1, N)
    sl = _int_blocks(slabels)
    cl = _int_blocks(clabels)
    bi = _int_blocks(batch_idx)

    int_spec = pl.BlockSpec((1, 1, NB_PTS), lambda j: (j, 0, 0))
    et_spec = pl.BlockSpec((D, NB_PTS), lambda j: (0, j))
    mt_spec = pl.BlockSpec((1, NB_PTS), lambda j: (0, j))

    stats = pl.pallas_call(
        _stats_kernel,
        grid=(NUM_BLK,),
        in_specs=[et_spec, mt_spec, int_spec, int_spec, int_spec],
        out_specs=pl.BlockSpec((NROW, 10), lambda j: (0, 0)),
        out_shape=jax.ShapeDtypeStruct((NROW, 10), jnp.float32),
    )(et, mt, sl, cl, bi)

    out = pl.pallas_call(
        _loss_kernel,
        grid=(NUM_BLK,),
        in_specs=[pl.BlockSpec((NROW, 10), lambda j: (0, 0)),
                  et_spec, mt_spec, int_spec, int_spec, int_spec],
        out_specs=pl.BlockSpec((1, 1), lambda j: (0, 0)),
        out_shape=jax.ShapeDtypeStruct((1, 1), jnp.float32),
        scratch_shapes=[pltpu.VMEM((NSEG, 1), jnp.float32),
                        pltpu.VMEM((NSEG, 1), jnp.float32)],
    )(stats, et, mt, sl, cl, bi)

    return out[0, 0]


# speedup vs baseline: 10.3569x; 1.4313x over previous
import functools

import jax
import jax.numpy as jnp
from jax import lax
from jax.experimental import pallas as pl
from jax.experimental.pallas import tpu as pltpu
from jax.experimental.pallas import tpu_sc as plsc

N = 131072
D = 8
NB_PTS = 1024
NUM_BLK = N // NB_PTS
NSEG = 512
NROW = 640
BROW = 544
NW = 32
PTS_W = N // NW
TBL = NROW * 16

XC_LO = 1.0000005000002917e-06
XC_HI = 13.815510557964274

_DN = (((1,), (1,)), ((), ()))
_DN2 = (((1,), (0,)), ((), ()))
_PREC = lax.Precision.HIGHEST

@functools.cache
def _get_sc_stats():
    mesh = plsc.VectorSubcoreMesh(core_axis_name="c", subcore_axis_name="s")
    return functools.partial(
        pl.kernel,
        mesh=mesh,
        out_type=jax.ShapeDtypeStruct((NW, TBL), jnp.float32),
        compiler_params=pltpu.CompilerParams(needs_layout_passes=False),
        scratch_types=[
            pltpu.VMEM((D, PTS_W), jnp.float32),
            pltpu.VMEM((PTS_W,), jnp.float32),
            pltpu.VMEM((PTS_W,), jnp.int32),
            pltpu.VMEM((PTS_W,), jnp.int32),
            pltpu.VMEM((PTS_W,), jnp.int32),
            pltpu.VMEM((TBL,), jnp.float32),
        ],
    )(_sc_stats_body)


def _sc_stats_body(et_hbm, mt_hbm, sl_hbm, cl_hbm, bi_hbm, zeros_hbm, out_hbm,
                   e_v, m_v, sl_v, cl_v, bi_v, tbl):
    cid = lax.axis_index("c")
    sub = lax.axis_index("s")
    wid = sub * 2 + cid
    base = wid * PTS_W

    pltpu.sync_copy(et_hbm.at[:, pl.ds(base, PTS_W)], e_v)
    pltpu.sync_copy(mt_hbm.at[pl.ds(base, PTS_W)], m_v)
    pltpu.sync_copy(sl_hbm.at[pl.ds(base, PTS_W)], sl_v)
    pltpu.sync_copy(cl_hbm.at[pl.ds(base, PTS_W)], cl_v)
    pltpu.sync_copy(bi_hbm.at[pl.ds(base, PTS_W)], bi_v)
    pltpu.sync_copy(zeros_hbm, tbl)

    ones = jnp.ones((16,), jnp.float32)

    def body(i, carry):
        off = i * 16
        sl = sl_v[pl.ds(off, 16)]
        cl = cl_v[pl.ds(off, 16)]
        bi = bi_v[pl.ds(off, 16)]
        seg = bi * 128 + sl * 32 + cl
        seg = jnp.where(sl < 4, seg, 560)
        addr = seg * 16
        for q in range(D):
            plsc.addupdate_scatter(tbl, [addr + q], e_v[q, pl.ds(off, 16)])
        plsc.addupdate_scatter(tbl, [addr + 8], m_v[pl.ds(off, 16)])
        plsc.addupdate_scatter(tbl, [addr + 9], ones)
        plsc.addupdate_scatter(tbl, [(bi + BROW) * 16 + 9], ones)
        return carry

    lax.fori_loop(0, PTS_W // 16, body, 0)

    pltpu.sync_copy(tbl, out_hbm.at[wid])


def _loss_kernel(tables_ref, et_ref, mt_ref, sl_ref, cl_ref, bi_ref,
                 out_ref, stats_s, bce_acc, sm_acc):
    step = pl.program_id(0)

    @pl.when(step == 0)
    def _():
        acc = tables_ref[0]
        for w in range(1, NW):
            acc = acc + tables_ref[w]
        stats_s[...] = acc
        bce_acc[...] = jnp.zeros_like(bce_acc)
        sm_acc[...] = jnp.zeros_like(sm_acc)

    cnt = stats_s[0:NSEG, 9:10]
    cnt_safe = jnp.maximum(cnt, 1.0)
    inv_cnt = 1.0 / cnt_safe
    cmat = stats_s[0:NSEG, 0:D] * inv_cnt
    sigma = stats_s[0:NSEG, 8:9] * inv_cnt
    inv_den = 1.0 / (2.0 * sigma * sigma + 1e-8)
    cnorm2 = jnp.sum(cmat * cmat, axis=1, keepdims=True)

    e = et_ref[...]
    m = mt_ref[...]
    sl = sl_ref[0]
    cl = cl_ref[0]
    bi = bi_ref[0]
    sid = bi * 128 + sl * 32 + cl
    gid = bi * 4 + sl

    enorm2 = jnp.sum(e * e, axis=0, keepdims=True)
    dotp = lax.dot_general(cmat, e, _DN2,
                           preferred_element_type=jnp.float32,
                           precision=_PREC)
    x = (cnorm2 - 2.0 * dotp + enorm2) * inv_den
    xc = jnp.clip(x, XC_LO, XC_HI)
    p = jnp.exp(-xc)
    log1mp = jnp.log1p(-p)

    rows = lax.broadcasted_iota(jnp.int32, (NSEG, NB_PTS), 0)
    tgt = rows == sid
    g_lo = gid * 32
    valid = (rows >= g_lo) & (rows < g_lo + 32) & (sl < 4)
    term = jnp.where(tgt, xc, -log1mp)
    term = jnp.where(valid, term, 0.0)
    bce_acc[...] += jnp.sum(term, axis=1, keepdims=True)

    dm = m - sigma
    smooth = jnp.where(tgt & valid, dm * dm, 0.0)
    sm_acc[...] += jnp.sum(smooth, axis=1, keepdims=True)

    @pl.when(step == NUM_BLK - 1)
    def _():
        gi = lax.broadcasted_iota(jnp.int32, (16, NSEG), 0)
        si = lax.broadcasted_iota(jnp.int32, (16, NSEG), 1)
        m1 = ((si // 32) == gi).astype(jnp.float32)
        bi4 = lax.broadcasted_iota(jnp.int32, (4, 16), 0)
        gi16 = lax.broadcasted_iota(jnp.int32, (4, 16), 1)
        m2 = ((gi16 // 4) == bi4).astype(jnp.float32)

        def gdot(mat, vec):
            return lax.dot_general(mat, vec, _DN2,
                                   preferred_element_type=jnp.float32,
                                   precision=_PREC)

        present = (cnt > 0.0).astype(jnp.float32)
        n_sel = gdot(m1, cnt)
        npres = gdot(m1, present)
        bce_g = gdot(m1, present * bce_acc[...])
        sm_g = gdot(m1, sm_acc[...])
        n_sel_safe = jnp.maximum(n_sel, 1.0)
        npres_safe = jnp.maximum(npres, 1.0)
        ml = bce_g / n_sel_safe / npres_safe
        sml = sm_g / npres_safe
        s_present = (n_sel > 0.0).astype(jnp.float32)
        contrib = s_present * (ml + sml)
        cls_sum = gdot(m2, contrib)
        cls_cnt = gdot(m2, s_present)
        batch_loss = cls_sum / jnp.maximum(cls_cnt, 1.0)
        bcnt = stats_s[BROW:BROW + 4, 9:10]
        b_present = (bcnt > 0.0).astype(jnp.float32)
        num = jnp.sum(b_present * batch_loss, keepdims=True)
        den = jnp.maximum(jnp.sum(b_present, keepdims=True), 1.0)
        out_ref[...] = num / den


def _int_blocks(x):
    return x.astype(jnp.int32).reshape(NUM_BLK, 1, NB_PTS)


@jax.jit
def kernel(embeddings, margins, slabels, clabels, batch_idx):
    et = embeddings.T
    mt = margins.reshape(1, N)
    sl32 = slabels.astype(jnp.int32)
    cl32 = clabels.astype(jnp.int32)
    bi32 = batch_idx.astype(jnp.int32)
    zeros = jnp.zeros((TBL,), jnp.float32)

    tables = _get_sc_stats()(et, mt.reshape(N), sl32, cl32, bi32, zeros)
    tables = tables.reshape(NW, NROW, 16)

    sl = sl32.reshape(NUM_BLK, 1, NB_PTS)
    cl = cl32.reshape(NUM_BLK, 1, NB_PTS)
    bi = bi32.reshape(NUM_BLK, 1, NB_PTS)

    int_spec = pl.BlockSpec((1, 1, NB_PTS), lambda j: (j, 0, 0))
    et_spec = pl.BlockSpec((D, NB_PTS), lambda j: (0, j))
    mt_spec = pl.BlockSpec((1, NB_PTS), lambda j: (0, j))

    out = pl.pallas_call(
        _loss_kernel,
        grid=(NUM_BLK,),
        in_specs=[pl.BlockSpec((NW, NROW, 16), lambda j: (0, 0, 0)),
                  et_spec, mt_spec, int_spec, int_spec, int_spec],
        out_specs=pl.BlockSpec((1, 1), lambda j: (0, 0)),
        out_shape=jax.ShapeDtypeStruct((1, 1), jnp.float32),
        scratch_shapes=[pltpu.VMEM((NROW, 16), jnp.float32),
                        pltpu.VMEM((NSEG, 1), jnp.float32),
                        pltpu.VMEM((NSEG, 1), jnp.float32)],
    )(tables, et, mt, sl, cl, bi)

    return out[0, 0]
